# baseline (speedup 1.0000x reference)
import jax
import jax.numpy as jnp
from jax import lax
from jax.experimental import pallas as pl
from jax.experimental.pallas import tpu as pltpu
from jax.experimental.pallas import tpu_sc as plsc

N = 10000
E = 320000
D = 128

NC = 2
NS = 16
NW = NC * NS

NPAD = 10240
ROWS_PER_TILE = NPAD // NS
C = 128
EPW = 10240
NCHUNK = EPW // C
NPH = 2
CPP = NCHUNK // NPH

_MESH = plsc.VectorSubcoreMesh(
    core_axis_name="c", subcore_axis_name="s", num_cores=NC, num_subcores=NS)


def _deg_body(dst_hbm, out_hbm, didx, ones, zbuf, dacc):
    c = lax.axis_index("c")
    s = lax.axis_index("s")
    wid = c * NS + s
    for i in range(C // 16):
        ones[pl.ds(i * 16, 16)] = jnp.ones((16,), jnp.float32)
    for i in range(ROWS_PER_TILE // 16):
        zbuf[pl.ds(i * 16, 16)] = jnp.zeros((16,), jnp.float32)
    base = s * ROWS_PER_TILE
    pltpu.sync_copy(zbuf, dacc.at[pl.ds(base, ROWS_PER_TILE)])
    pltpu.sync_copy(dst_hbm.at[wid], didx)
    plsc.subcore_barrier()

    def body(i, carry):
        pltpu.sync_copy(ones, dacc.at[didx.at[i]], add=True)
        return carry

    lax.fori_loop(0, NCHUNK, body, 0)
    plsc.subcore_barrier()
    pltpu.sync_copy(dacc.at[pl.ds(base, ROWS_PER_TILE)],
                    out_hbm.at[c, pl.ds(base, ROWS_PER_TILE)])


_deg_kernel = pl.kernel(
    _deg_body,
    out_type=jax.ShapeDtypeStruct((NC, NPAD), jnp.float32),
    mesh=_MESH,
    scratch_types=[
        pltpu.VMEM((NCHUNK, C), jnp.int32),
        pltpu.VMEM((C,), jnp.float32),
        pltpu.VMEM((ROWS_PER_TILE,), jnp.float32),
        pltpu.VMEM_SHARED((NPAD,), jnp.float32),
    ],
)


def _scatter_body(g_hbm, src_hbm, dst_hbm, out_hbm,
                  sidx, didx, rows, zbuf, acc, sem0, sem1):
    c = lax.axis_index("c")
    s = lax.axis_index("s")
    wid = c * NS + s
    base = s * ROWS_PER_TILE
    for i in range(32 * D // 16):
        zbuf[i // 8, pl.ds((i % 8) * 16, 16)] = jnp.zeros((16,), jnp.float32)
    for r in range(ROWS_PER_TILE // 32):
        pltpu.sync_copy(zbuf, acc.at[pl.ds(base + r * 32, 32)])
    pltpu.sync_copy(src_hbm.at[wid, pl.ds(0, CPP)], sidx)
    pltpu.sync_copy(dst_hbm.at[wid, pl.ds(0, CPP)], didx)
    plsc.subcore_barrier()

    def start(i, b, sem):
        pltpu.async_copy(g_hbm.at[sidx.at[i]], rows.at[b], sem)

    def wait(b, sem):
        pltpu.make_async_copy(g_hbm.at[sidx.at[0]], rows.at[b], sem).wait()

    for p in range(NPH):
        if p > 0:
            pltpu.sync_copy(src_hbm.at[wid, pl.ds(p * CPP, CPP)], sidx)
            pltpu.sync_copy(dst_hbm.at[wid, pl.ds(p * CPP, CPP)], didx)
        start(0, 0, sem0)

        def body(k, carry):
            i0 = 2 * k
            start(i0 + 1, 1, sem1)
            wait(0, sem0)
            pltpu.sync_copy(rows.at[0], acc.at[didx.at[i0]], add=True)

            @pl.when(k < CPP // 2 - 1)
            def _():
                start(i0 + 2, 0, sem0)

            wait(1, sem1)
            pltpu.sync_copy(rows.at[1], acc.at[didx.at[i0 + 1]], add=True)
            return carry

        lax.fori_loop(0, CPP // 2, body, 0)
    plsc.subcore_barrier()
    pltpu.sync_copy(acc.at[pl.ds(base, ROWS_PER_TILE)],
                    out_hbm.at[c, pl.ds(base, ROWS_PER_TILE)])


_scatter_kernel = pl.kernel(
    _scatter_body,
    out_type=jax.ShapeDtypeStruct((NC, NPAD, D), jnp.float32),
    mesh=_MESH,
    scratch_types=[
        pltpu.VMEM((CPP, C), jnp.int32),
        pltpu.VMEM((CPP, C), jnp.int32),
        pltpu.VMEM((2, C, D), jnp.float32),
        pltpu.VMEM((32, D), jnp.float32),
        pltpu.VMEM_SHARED((NPAD, D), jnp.float32),
        pltpu.SemaphoreType.DMA,
        pltpu.SemaphoreType.DMA,
    ],
)


def _tc_mm_body(x_ref, w_ref, out_ref):
    out_ref[:N] = jnp.dot(x_ref[:], w_ref[:],
                          preferred_element_type=jnp.float32)
    out_ref[N:] = jnp.zeros((NPAD - N, D), jnp.float32)


def _tc_scale_body(h_ref, dinv_ref, out_ref):
    out_ref[:N] = h_ref[:N] * dinv_ref[:]
    out_ref[N:] = jnp.zeros((NPAD - N, D), jnp.float32)


def _tc_mid_body(agg_ref, g_ref, dinv_ref, b_ref, gamma_ref, beta_ref,
                 w_ref, out_ref):
    h = (agg_ref[0] + agg_ref[1] + g_ref[:]) * dinv_ref[:] + b_ref[:]
    m = jnp.mean(h, axis=0, keepdims=True)
    v = jnp.mean((h - m) ** 2, axis=0, keepdims=True)
    hn = (h - m) * lax.rsqrt(v + 1e-5) * gamma_ref[:] + beta_ref[:]
    r = jnp.maximum(hn, 0.0)
    h2 = jnp.dot(r, w_ref[:], preferred_element_type=jnp.float32)
    out_ref[:N] = h2 * dinv_ref[:]
    out_ref[N:] = jnp.zeros((NPAD - N, D), jnp.float32)


def _tc_out_body(agg_ref, g_ref, dinv_ref, b_ref, out_ref):
    out_ref[:] = (agg_ref[0] + agg_ref[1] + g_ref[:]) * dinv_ref[:] + b_ref[:]


_tc_mm = pl.pallas_call(
    _tc_mm_body,
    out_shape=jax.ShapeDtypeStruct((NPAD, D), jnp.float32),
)

_tc_scale = pl.pallas_call(
    _tc_scale_body,
    out_shape=jax.ShapeDtypeStruct((NPAD, D), jnp.float32),
)

_tc_mid = pl.pallas_call(
    _tc_mid_body,
    out_shape=jax.ShapeDtypeStruct((NPAD, D), jnp.float32),
)

_tc_out = pl.pallas_call(
    _tc_out_body,
    out_shape=jax.ShapeDtypeStruct((N, D), jnp.float32),
)


def kernel(x, edge_index, W1, b1, gamma1, beta1, W2, b2, gamma2, beta2,
           W3, b3):
    src = edge_index[0]
    dst = edge_index[1]
    npad_edges = EPW * NW - E
    it = jnp.arange(npad_edges, dtype=jnp.int32)
    src_p = jnp.concatenate([src, it % N]).reshape(NW, NCHUNK, C)
    dst_p = jnp.concatenate([dst, N + it % (NPAD - N)]).reshape(NW, NCHUNK, C)

    h1 = _tc_mm(x, W1)
    deg2 = _deg_kernel(dst_p)
    deg = deg2[0, :N] + deg2[1, :N] + 1.0
    dinv = lax.rsqrt(deg).reshape(N, 1)

    g1 = _tc_scale(h1, dinv)
    agg1 = _scatter_kernel(g1, src_p, dst_p)
    g2 = _tc_mid(agg1[:, :N], g1[:N], dinv, b1, gamma1, beta1, W2)
    agg2 = _scatter_kernel(g2, src_p, dst_p)
    g3 = _tc_mid(agg2[:, :N], g2[:N], dinv, b2, gamma2, beta2, W3)
    agg3 = _scatter_kernel(g3, src_p, dst_p)
    return _tc_out(agg3[:, :N], g3[:N], dinv, b3)


# speedup vs baseline: 1.0019x; 1.0019x over previous
import jax
import jax.numpy as jnp
from jax import lax
from jax.experimental import pallas as pl
from jax.experimental.pallas import tpu as pltpu
from jax.experimental.pallas import tpu_sc as plsc

N = 10000
E = 320000
D = 128

NC = 2
NS = 16
NW = NC * NS

NPAD = 10240
ROWS_PER_TILE = NPAD // NS
C = 128
EPW = 10240
NCHUNK = EPW // C
NPH = 2
CPP = NCHUNK // NPH

_MESH = plsc.VectorSubcoreMesh(
    core_axis_name="c", subcore_axis_name="s", num_cores=NC, num_subcores=NS)


def _deg_body(dst_hbm, out_hbm, didx, ones, zbuf, dacc):
    c = lax.axis_index("c")
    s = lax.axis_index("s")
    wid = c * NS + s
    for i in range(C // 16):
        ones[pl.ds(i * 16, 16)] = jnp.ones((16,), jnp.float32)
    for i in range(ROWS_PER_TILE // 16):
        zbuf[pl.ds(i * 16, 16)] = jnp.zeros((16,), jnp.float32)
    base = s * ROWS_PER_TILE
    pltpu.sync_copy(zbuf, dacc.at[pl.ds(base, ROWS_PER_TILE)])
    pltpu.sync_copy(dst_hbm.at[wid], didx)
    plsc.subcore_barrier()

    def body(i, carry):
        pltpu.sync_copy(ones, dacc.at[didx.at[i]], add=True)
        return carry

    lax.fori_loop(0, NCHUNK, body, 0)
    plsc.subcore_barrier()
    pltpu.sync_copy(dacc.at[pl.ds(base, ROWS_PER_TILE)],
                    out_hbm.at[c, pl.ds(base, ROWS_PER_TILE)])


_deg_kernel = pl.kernel(
    _deg_body,
    out_type=jax.ShapeDtypeStruct((NC, NPAD), jnp.float32),
    mesh=_MESH,
    scratch_types=[
        pltpu.VMEM((NCHUNK, C), jnp.int32),
        pltpu.VMEM((C,), jnp.float32),
        pltpu.VMEM((ROWS_PER_TILE,), jnp.float32),
        pltpu.VMEM_SHARED((NPAD,), jnp.float32),
    ],
)


def _scatter_body(g_hbm, src_hbm, dst_hbm, out_hbm,
                  sidx, didx, rows, zbuf, acc, sem0, sem1):
    c = lax.axis_index("c")
    s = lax.axis_index("s")
    wid = c * NS + s
    base = s * ROWS_PER_TILE
    for i in range(32 * D // 16):
        zbuf[i // 8, pl.ds((i % 8) * 16, 16)] = jnp.zeros((16,), jnp.float32)
    for r in range(ROWS_PER_TILE // 32):
        pltpu.sync_copy(zbuf, acc.at[pl.ds(base + r * 32, 32)])
    pltpu.sync_copy(src_hbm.at[wid, pl.ds(0, CPP)], sidx)
    pltpu.sync_copy(dst_hbm.at[wid, pl.ds(0, CPP)], didx)
    plsc.subcore_barrier()

    def start(i, b, sem):
        pltpu.async_copy(g_hbm.at[sidx.at[i, pl.ds(0, C // 2)]],
                         rows.at[b, pl.ds(0, C // 2)], sem)
        pltpu.async_copy(g_hbm.at[sidx.at[i, pl.ds(C // 2, C // 2)]],
                         rows.at[b, pl.ds(C // 2, C // 2)], sem)

    def wait(b, sem):
        pltpu.make_async_copy(g_hbm.at[sidx.at[0]], rows.at[b], sem).wait()

    for p in range(NPH):
        if p > 0:
            pltpu.sync_copy(src_hbm.at[wid, pl.ds(p * CPP, CPP)], sidx)
            pltpu.sync_copy(dst_hbm.at[wid, pl.ds(p * CPP, CPP)], didx)
        start(0, 0, sem0)

        def body(k, carry):
            i0 = 2 * k
            start(i0 + 1, 1, sem1)
            wait(0, sem0)
            pltpu.sync_copy(rows.at[0], acc.at[didx.at[i0]], add=True)

            @pl.when(k < CPP // 2 - 1)
            def _():
                start(i0 + 2, 0, sem0)

            wait(1, sem1)
            pltpu.sync_copy(rows.at[1], acc.at[didx.at[i0 + 1]], add=True)
            return carry

        lax.fori_loop(0, CPP // 2, body, 0)
    plsc.subcore_barrier()
    pltpu.sync_copy(acc.at[pl.ds(base, ROWS_PER_TILE)],
                    out_hbm.at[c, pl.ds(base, ROWS_PER_TILE)])


_scatter_kernel = pl.kernel(
    _scatter_body,
    out_type=jax.ShapeDtypeStruct((NC, NPAD, D), jnp.float32),
    mesh=_MESH,
    scratch_types=[
        pltpu.VMEM((CPP, C), jnp.int32),
        pltpu.VMEM((CPP, C), jnp.int32),
        pltpu.VMEM((2, C, D), jnp.float32),
        pltpu.VMEM((32, D), jnp.float32),
        pltpu.VMEM_SHARED((NPAD, D), jnp.float32),
        pltpu.SemaphoreType.DMA,
        pltpu.SemaphoreType.DMA,
    ],
)


def _tc_in_body(x_ref, w_ref, dinv_ref, out_ref):
    h = jnp.dot(x_ref[:], w_ref[:], preferred_element_type=jnp.float32)
    out_ref[:N] = h * dinv_ref[:]
    out_ref[N:] = jnp.zeros((NPAD - N, D), jnp.float32)


def _tc_mid_body(agg_ref, g_ref, dinv_ref, b_ref, gamma_ref, beta_ref,
                 w_ref, out_ref):
    h = (agg_ref[0] + agg_ref[1] + g_ref[:]) * dinv_ref[:] + b_ref[:]
    m = jnp.mean(h, axis=0, keepdims=True)
    v = jnp.mean((h - m) ** 2, axis=0, keepdims=True)
    hn = (h - m) * lax.rsqrt(v + 1e-5) * gamma_ref[:] + beta_ref[:]
    r = jnp.maximum(hn, 0.0)
    h2 = jnp.dot(r, w_ref[:], preferred_element_type=jnp.float32)
    out_ref[:N] = h2 * dinv_ref[:]
    out_ref[N:] = jnp.zeros((NPAD - N, D), jnp.float32)


def _tc_out_body(agg_ref, g_ref, dinv_ref, b_ref, out_ref):
    out_ref[:] = (agg_ref[0] + agg_ref[1] + g_ref[:]) * dinv_ref[:] + b_ref[:]


_tc_in = pl.pallas_call(
    _tc_in_body,
    out_shape=jax.ShapeDtypeStruct((NPAD, D), jnp.float32),
)

_tc_mid = pl.pallas_call(
    _tc_mid_body,
    out_shape=jax.ShapeDtypeStruct((NPAD, D), jnp.float32),
)

_tc_out = pl.pallas_call(
    _tc_out_body,
    out_shape=jax.ShapeDtypeStruct((N, D), jnp.float32),
)


def kernel(x, edge_index, W1, b1, gamma1, beta1, W2, b2, gamma2, beta2,
           W3, b3):
    src = edge_index[0]
    dst = edge_index[1]
    npad_edges = EPW * NW - E
    it = jnp.arange(npad_edges, dtype=jnp.int32)
    src_p = jnp.concatenate([src, it % N]).reshape(NW, NCHUNK, C)
    dst_p = jnp.concatenate([dst, N + it % (NPAD - N)]).reshape(NW, NCHUNK, C)

    deg2 = _deg_kernel(dst_p)
    deg = deg2[0, :N] + deg2[1, :N] + 1.0
    dinv = lax.rsqrt(deg).reshape(N, 1)

    g1 = _tc_in(x, W1, dinv)
    agg1 = _scatter_kernel(g1, src_p, dst_p)
    g2 = _tc_mid(agg1[:, :N], g1[:N], dinv, b1, gamma1, beta1, W2)
    agg2 = _scatter_kernel(g2, src_p, dst_p)
    g3 = _tc_mid(agg2[:, :N], g2[:N], dinv, b2, gamma2, beta2, W3)
    agg3 = _scatter_kernel(g3, src_p, dst_p)
    return _tc_out(agg3[:, :N], g3[:N], dinv, b3)
